# baseline (speedup 1.0000x reference)
import functools

import jax
import jax.numpy as jnp
from jax import lax
from jax.experimental import pallas as pl
from jax.experimental.pallas import tpu as pltpu
from jax.experimental.pallas import tpu_sc as plsc

B, N, L, D = 8, 4, 2048, 768
G = 320
NUM_TILES = 32
TOK = B * L
TPT = TOK // NUM_TILES
BL = 512
NJ = L // BL


def _sc_gather(vq_indices, weight, lens):
    mesh = plsc.VectorSubcoreMesh(core_axis_name="c", subcore_axis_name="s")

    @functools.partial(
        pl.kernel,
        out_type=jax.ShapeDtypeStruct((B, NJ, 1, BL), jnp.float32),
        mesh=mesh,
        scratch_types=[
            pltpu.VMEM_SHARED((G, G), jnp.float32),
            pltpu.VMEM((G, G), jnp.float32),
            pltpu.VMEM((TPT, 2), jnp.int32),
            pltpu.VMEM((TPT,), jnp.float32),
            pltpu.VMEM((8,), jnp.int32),
            pltpu.SemaphoreType.DMA,
            pltpu.SemaphoreType.DMA,
        ],
        compiler_params=pltpu.CompilerParams(
            needs_layout_passes=False, use_tc_tiling_on_sc=False),
    )
    def k(vq_hbm, wt_hbm, len_hbm, w_hbm, table_sh, table_v, idx_v, w_v,
          len_v, sem0, sem1):
        sid = lax.axis_index("s")
        wid = sid * 2 + lax.axis_index("c")
        b = wid // NJ
        jblk = wid % NJ
        l0 = jblk * TPT

        cp1 = pltpu.make_async_copy(
            vq_hbm.at[b, pl.ds(l0, TPT)], idx_v, sem1)
        cp1.start()
        pltpu.sync_copy(len_hbm, len_v)

        @pl.when(sid == 0)
        def _():
            pltpu.sync_copy(wt_hbm, table_sh)
        plsc.subcore_barrier()
        cp0 = pltpu.make_async_copy(table_sh, table_v, sem0)
        cp0.start()

        lenb = plsc.load_gather(len_v, [jnp.full((16,), b, jnp.int32)])
        iot = lax.iota(jnp.int32, 16)
        zero16 = jnp.zeros((16,), jnp.int32)
        one16 = jnp.ones((16,), jnp.int32)
        cp1.wait()
        cp0.wait()
        for j in range(TPT // 16):
            rows = j * 16 + iot
            i0 = plsc.load_gather(idx_v, [rows, zero16])
            i1 = plsc.load_gather(idx_v, [rows, one16])
            wv = plsc.load_gather(table_v, [i0, i1])
            pos = l0 + rows
            wv = jnp.where(pos < lenb, wv, jnp.zeros_like(wv))
            w_v[pl.ds(j * 16, 16)] = wv
        pltpu.sync_copy(w_v, w_hbm.at[b, jblk, 0])

    return k(vq_indices, weight, lens)


def _tc_reduce(x_full, w4, lens):
    def body(lens_ref, w_ref, x_ref, o_ref):
        b = pl.program_id(0)
        j = pl.program_id(1)

        @pl.when((b == 0) & (j == 0))
        def _():
            o_ref[...] = jnp.zeros_like(o_ref)

        @pl.when(j * BL < lens_ref[b])
        def _():
            wv = w_ref[b, j]
            xm = x_ref[0, 0]
            o_ref[pl.ds(b, 1), :] += lax.dot_general(
                wv, xm, (((1,), (0,)), ((), ())),
                preferred_element_type=jnp.float32)

    def x_map(b, j, lens):
        jmax = jnp.maximum((lens[b] + BL - 1) // BL - 1, 0)
        return (b, N - 1, jnp.minimum(j, jmax), 0)

    grid_spec = pltpu.PrefetchScalarGridSpec(
        num_scalar_prefetch=1,
        grid=(B, NJ),
        in_specs=[
            pl.BlockSpec((B, NJ, 1, BL), lambda b, j, lens: (0, 0, 0, 0)),
            pl.BlockSpec((1, 1, BL, D), x_map),
        ],
        out_specs=pl.BlockSpec((B, D), lambda b, j, lens: (0, 0)),
    )
    return pl.pallas_call(
        body,
        grid_spec=grid_spec,
        out_shape=jax.ShapeDtypeStruct((B, D), jnp.float32),
        compiler_params=pltpu.CompilerParams(
            dimension_semantics=("arbitrary", "arbitrary")),
    )(lens, w4, x_full)


def kernel(input_feature, input_lengths, vq_indices, weight):
    lens = input_lengths.astype(jnp.int32)
    w4 = _sc_gather(vq_indices, weight, lens)
    return _tc_reduce(input_feature, w4, lens)


# speedup vs baseline: 1.2392x; 1.2392x over previous
import functools

import jax
import jax.numpy as jnp
from jax import lax
from jax.experimental import pallas as pl
from jax.experimental.pallas import tpu as pltpu
from jax.experimental.pallas import tpu_sc as plsc

B, N, L, D = 8, 4, 2048, 768
G = 320
NUM_TILES = 32
TOK = B * L
TPT = TOK // NUM_TILES
BL = 512
NJ = L // BL


def _sc_gather(vq_indices, weight, lens):
    mesh = plsc.VectorSubcoreMesh(core_axis_name="c", subcore_axis_name="s")

    @functools.partial(
        pl.kernel,
        out_type=jax.ShapeDtypeStruct((B, NJ, 1, BL), jnp.float32),
        mesh=mesh,
        scratch_types=[
            pltpu.VMEM_SHARED((G, G), jnp.float32),
            pltpu.VMEM((G, G), jnp.float32),
            pltpu.VMEM((TPT,), jnp.int32),
            pltpu.VMEM((TPT,), jnp.float32),
            pltpu.VMEM((8,), jnp.int32),
            pltpu.SemaphoreType.DMA,
            pltpu.SemaphoreType.DMA,
        ],
        compiler_params=pltpu.CompilerParams(
            needs_layout_passes=False, use_tc_tiling_on_sc=False),
    )
    def k(vq_hbm, wt_hbm, len_hbm, w_hbm, table_sh, table_v, idx_v, w_v,
          len_v, sem0, sem1):
        sid = lax.axis_index("s")
        wid = sid * 2 + lax.axis_index("c")
        b = wid // NJ
        jblk = wid % NJ
        l0 = jblk * TPT

        cp1 = pltpu.make_async_copy(
            vq_hbm.at[pl.ds(wid * TPT, TPT)], idx_v, sem1)
        cp1.start()
        pltpu.sync_copy(len_hbm, len_v)

        @pl.when(sid == 0)
        def _():
            pltpu.sync_copy(wt_hbm, table_sh)
        plsc.subcore_barrier()
        cp0 = pltpu.make_async_copy(table_sh, table_v, sem0)
        cp0.start()

        lenb = plsc.load_gather(len_v, [jnp.full((16,), b, jnp.int32)])
        iot = lax.iota(jnp.int32, 16)
        cp1.wait()
        cp0.wait()
        for j in range(TPT // 16):
            rows = j * 16 + iot
            fv = plsc.load_gather(idx_v, [rows])
            i0 = fv // G
            i1 = fv - i0 * G
            wv = plsc.load_gather(table_v, [i0, i1])
            pos = l0 + rows
            wv = jnp.where(pos < lenb, wv, jnp.zeros_like(wv))
            w_v[pl.ds(j * 16, 16)] = wv
        pltpu.sync_copy(w_v, w_hbm.at[b, jblk, 0])

    return k(vq_indices, weight, lens)


def _tc_reduce(x_full, w4, lens):
    def body(lens_ref, w_ref, x_ref, o_ref):
        b = pl.program_id(0)
        j = pl.program_id(1)

        @pl.when((b == 0) & (j == 0))
        def _():
            o_ref[...] = jnp.zeros_like(o_ref)

        @pl.when(j * BL < lens_ref[b])
        def _():
            wv = w_ref[b, j]
            xm = x_ref[0, 0]
            o_ref[pl.ds(b, 1), :] += lax.dot_general(
                wv, xm, (((1,), (0,)), ((), ())),
                preferred_element_type=jnp.float32)

    def x_map(b, j, lens):
        jmax = jnp.maximum((lens[b] + BL - 1) // BL - 1, 0)
        return (b, N - 1, jnp.minimum(j, jmax), 0)

    grid_spec = pltpu.PrefetchScalarGridSpec(
        num_scalar_prefetch=1,
        grid=(B, NJ),
        in_specs=[
            pl.BlockSpec((B, NJ, 1, BL), lambda b, j, lens: (0, 0, 0, 0)),
            pl.BlockSpec((1, 1, BL, D), x_map),
        ],
        out_specs=pl.BlockSpec((B, D), lambda b, j, lens: (0, 0)),
    )
    return pl.pallas_call(
        body,
        grid_spec=grid_spec,
        out_shape=jax.ShapeDtypeStruct((B, D), jnp.float32),
        compiler_params=pltpu.CompilerParams(
            dimension_semantics=("arbitrary", "arbitrary")),
    )(lens, w4, x_full)


def kernel(input_feature, input_lengths, vq_indices, weight):
    lens = input_lengths.astype(jnp.int32)
    fidx = (vq_indices[..., 0] * G + vq_indices[..., 1]).reshape(-1)
    w4 = _sc_gather(fidx, weight, lens)
    return _tc_reduce(input_feature, w4, lens)
